# baseline (speedup 1.0000x reference)
import jax
import jax.numpy as jnp
from jax.experimental import pallas as pl
from jax.experimental.pallas import tpu as pltpu

NSEL = 384
TB = 512


def _score_kernel(x_ref, w_ref, b_ref, o_ref):
    x = x_ref[...]
    w = w_ref[...]
    s = jax.lax.dot_general(x, w, (((1,), (1,)), ((), ())))
    o_ref[...] = jax.nn.sigmoid(s + b_ref[...])


def kernel(data, W, b):
    B, S, H = data.shape
    N = B * S
    x = data.reshape(N, H)
    b2 = b.reshape(1, H)
    imp = pl.pallas_call(
        _score_kernel,
        grid=(N // TB,),
        in_specs=[
            pl.BlockSpec((TB, H), lambda i: (i, 0)),
            pl.BlockSpec((H, H), lambda i: (0, 0)),
            pl.BlockSpec((1, H), lambda i: (0, 0)),
        ],
        out_specs=pl.BlockSpec((TB, H), lambda i: (i, 0)),
        out_shape=jax.ShapeDtypeStruct((N, H), jnp.float32),
    )(x, W, b2)
    _, idx = jax.lax.top_k(imp, NSEL)
    sel = jnp.take_along_axis(x, idx, axis=-1)
    return sel.reshape(B, S, NSEL)
